# baseline (speedup 1.0000x reference)
import functools

import jax
import jax.numpy as jnp
from jax import lax
from jax.experimental import pallas as pl
from jax.experimental.pallas import tpu as pltpu
from jax.experimental.pallas import tpu_sc as plsc

_N = 16 * 3 * 512 * 512
_ROWS = 24576
_COLS = 512

_NC = 2
_NS = 16
_NW = _NC * _NS
_SC_ROWS = 8192
_W_ROWS = _SC_ROWS // _NW
_CH = 32
_NCH = _W_ROWS // _CH

_TC_ROWS = _ROWS - _SC_ROWS
_K = 4
_STEPS = 8
_BR = _TC_ROWS // (_K * _STEPS)
_TC_BASE = _SC_ROWS // _BR


def _tc_kernel(*refs):
    a_refs = refs[:_K]
    b_refs = refs[_K:2 * _K]
    o_ref = refs[2 * _K]
    acc_ref = refs[2 * _K + 1]
    i = pl.program_id(0)

    total = jnp.zeros((8, _COLS), jnp.float32)
    for k in range(_K):
        d = jnp.abs(a_refs[k][...] - b_refs[k][...])
        total = total + jnp.sum(d.reshape(_BR // 8, 8, _COLS), axis=0)

    @pl.when(i == 0)
    def _init():
        acc_ref[...] = total

    @pl.when(i > 0)
    def _acc():
        acc_ref[...] += total

    @pl.when(i == _STEPS - 1)
    def _fin():
        o_ref[0, 0] = jnp.sum(acc_ref[...]) * (1.0 / _N)


def _make_tc_spec(k):
    return pl.BlockSpec(
        (_BR, _COLS), lambda i, k=k: (_TC_BASE + k * _STEPS + i, 0)
    )


def _tc_partial(a, b):
    in_specs = [_make_tc_spec(k) for k in range(_K)] * 2
    out = pl.pallas_call(
        _tc_kernel,
        grid=(_STEPS,),
        in_specs=in_specs,
        out_specs=pl.BlockSpec(
            (1, 1), lambda i: (0, 0), memory_space=pltpu.SMEM
        ),
        out_shape=jax.ShapeDtypeStruct((1, 1), jnp.float32),
        scratch_shapes=[pltpu.VMEM((8, _COLS), jnp.float32)],
    )(*([a] * _K + [b] * _K))
    return out[0, 0]


def _sc_partial(a, b):
    mesh = plsc.VectorSubcoreMesh(
        core_axis_name="c", subcore_axis_name="s",
        num_cores=_NC, num_subcores=_NS,
    )

    @functools.partial(
        pl.kernel,
        out_type=jax.ShapeDtypeStruct((_NW, 16), jnp.float32),
        mesh=mesh,
        scratch_types=[
            pltpu.VMEM((_CH, _COLS), jnp.float32),
            pltpu.VMEM((_CH, _COLS), jnp.float32),
            pltpu.VMEM((_CH, _COLS), jnp.float32),
            pltpu.VMEM((_CH, _COLS), jnp.float32),
            pltpu.VMEM((16,), jnp.float32),
            pltpu.SemaphoreType.DMA,
            pltpu.SemaphoreType.DMA,
            pltpu.SemaphoreType.DMA,
            pltpu.SemaphoreType.DMA,
        ],
    )
    def sc_kernel(a_hbm, b_hbm, o_hbm, a0, a1, b0, b1, accv, s0, s1, s2, s3):
        wid = lax.axis_index("s") * _NC + lax.axis_index("c")
        base = wid * _W_ROWS
        abufs = (a0, a1)
        bbufs = (b0, b1)
        asems = (s0, s1)
        bsems = (s2, s3)

        def start(ci, slot):
            r0 = base + ci * _CH
            ca = pltpu.async_copy(
                a_hbm.at[pl.ds(r0, _CH)], abufs[slot], asems[slot]
            )
            cb = pltpu.async_copy(
                b_hbm.at[pl.ds(r0, _CH)], bbufs[slot], bsems[slot]
            )
            return ca, cb

        def accum_chunk(a_ref, b_ref, accs):
            def body(r, accs):
                new = list(accs)
                for t in range(32):
                    va = a_ref[r, pl.ds(t * 16, 16)]
                    vb = b_ref[r, pl.ds(t * 16, 16)]
                    new[t % 8] = new[t % 8] + jnp.abs(va - vb)
                return tuple(new)

            return plsc.parallel_loop(0, _CH, 1, unroll=2, carry=accs)(body)

        zero = jnp.zeros((16,), jnp.float32)
        accs = (zero,) * 8
        cur = start(0, 0)
        for ci in range(_NCH):
            slot = ci % 2
            nxt = start(ci + 1, (ci + 1) % 2) if ci + 1 < _NCH else None
            cur[0].wait()
            cur[1].wait()
            accs = accum_chunk(abufs[slot], bbufs[slot], accs)
            cur = nxt

        accv[...] = (
            ((accs[0] + accs[1]) + (accs[2] + accs[3]))
            + ((accs[4] + accs[5]) + (accs[6] + accs[7]))
        )
        pltpu.sync_copy(accv, o_hbm.at[wid])

    return sc_kernel(a, b)


def kernel(inp, tar, boxes, texts):
    a = inp.reshape(_ROWS, _COLS)
    b = tar.reshape(_ROWS, _COLS)
    sc_out = _sc_partial(a, b)
    tc_out = _tc_partial(a, b)
    return tc_out + jnp.sum(sc_out) * (1.0 / _N)


# speedup vs baseline: 1.0051x; 1.0051x over previous
import functools

import jax
import jax.numpy as jnp
from jax import lax
from jax.experimental import pallas as pl
from jax.experimental.pallas import tpu as pltpu
from jax.experimental.pallas import tpu_sc as plsc

_N = 16 * 3 * 512 * 512
_ROWS = 24576
_COLS = 512

_NC = 2
_NS = 16
_NW = _NC * _NS
_SC_ROWS = 8192
_W_ROWS = _SC_ROWS // _NW
_CH = 32
_NCH = _W_ROWS // _CH
_NBUF = 3

_TC_ROWS = _ROWS - _SC_ROWS
_K = 4
_STEPS = 8
_BR = _TC_ROWS // (_K * _STEPS)
_TC_BASE = _SC_ROWS // _BR


def _tc_kernel(*refs):
    a_refs = refs[:_K]
    b_refs = refs[_K:2 * _K]
    o_ref = refs[2 * _K]
    acc_ref = refs[2 * _K + 1]
    i = pl.program_id(0)

    total = jnp.zeros((8, _COLS), jnp.float32)
    for k in range(_K):
        d = jnp.abs(a_refs[k][...] - b_refs[k][...])
        total = total + jnp.sum(d.reshape(_BR // 8, 8, _COLS), axis=0)

    @pl.when(i == 0)
    def _init():
        acc_ref[...] = total

    @pl.when(i > 0)
    def _acc():
        acc_ref[...] += total

    @pl.when(i == _STEPS - 1)
    def _fin():
        o_ref[0, 0] = jnp.sum(acc_ref[...]) * (1.0 / _N)


def _make_tc_spec(k):
    return pl.BlockSpec(
        (_BR, _COLS), lambda i, k=k: (_TC_BASE + k * _STEPS + i, 0)
    )


def _tc_partial(a, b):
    in_specs = [_make_tc_spec(k) for k in range(_K)] * 2
    out = pl.pallas_call(
        _tc_kernel,
        grid=(_STEPS,),
        in_specs=in_specs,
        out_specs=pl.BlockSpec(
            (1, 1), lambda i: (0, 0), memory_space=pltpu.SMEM
        ),
        out_shape=jax.ShapeDtypeStruct((1, 1), jnp.float32),
        scratch_shapes=[pltpu.VMEM((8, _COLS), jnp.float32)],
    )(*([a] * _K + [b] * _K))
    return out[0, 0]


def _sc_partial(a, b):
    mesh = plsc.VectorSubcoreMesh(
        core_axis_name="c", subcore_axis_name="s",
        num_cores=_NC, num_subcores=_NS,
    )

    buf_types = [pltpu.VMEM((_CH, _COLS), jnp.float32)] * (2 * _NBUF)
    sem_types = [pltpu.SemaphoreType.DMA] * (2 * _NBUF)

    @functools.partial(
        pl.kernel,
        out_type=jax.ShapeDtypeStruct((_NW, 16), jnp.float32),
        mesh=mesh,
        scratch_types=buf_types + [pltpu.VMEM((16,), jnp.float32)] + sem_types,
    )
    def sc_kernel(a_hbm, b_hbm, o_hbm, *rest):
        abufs = rest[:_NBUF]
        bbufs = rest[_NBUF:2 * _NBUF]
        accv = rest[2 * _NBUF]
        asems = rest[2 * _NBUF + 1:2 * _NBUF + 1 + _NBUF]
        bsems = rest[2 * _NBUF + 1 + _NBUF:2 * _NBUF + 1 + 2 * _NBUF]
        wid = lax.axis_index("s") * _NC + lax.axis_index("c")
        base = wid * _W_ROWS

        def start(ci):
            slot = ci % _NBUF
            r0 = base + ci * _CH
            ca = pltpu.async_copy(
                a_hbm.at[pl.ds(r0, _CH)], abufs[slot], asems[slot]
            )
            cb = pltpu.async_copy(
                b_hbm.at[pl.ds(r0, _CH)], bbufs[slot], bsems[slot]
            )
            return ca, cb

        def accum_chunk(a_ref, b_ref, accs):
            def body(r, accs):
                new = list(accs)
                for t in range(32):
                    va = a_ref[r, pl.ds(t * 16, 16)]
                    vb = b_ref[r, pl.ds(t * 16, 16)]
                    new[t % 8] = new[t % 8] + jnp.abs(va - vb)
                return tuple(new)

            return plsc.parallel_loop(0, _CH, 1, unroll=2, carry=accs)(body)

        zero = jnp.zeros((16,), jnp.float32)
        accs = (zero,) * 8
        pending = [start(ci) for ci in range(min(_NBUF - 1, _NCH))]
        for ci in range(_NCH):
            if ci + _NBUF - 1 < _NCH:
                pending.append(start(ci + _NBUF - 1))
            ca, cb = pending.pop(0)
            ca.wait()
            cb.wait()
            slot = ci % _NBUF
            accs = accum_chunk(abufs[slot], bbufs[slot], accs)

        accv[...] = (
            ((accs[0] + accs[1]) + (accs[2] + accs[3]))
            + ((accs[4] + accs[5]) + (accs[6] + accs[7]))
        )
        pltpu.sync_copy(accv, o_hbm.at[wid])

    return sc_kernel(a, b)


def kernel(inp, tar, boxes, texts):
    a = inp.reshape(_ROWS, _COLS)
    b = tar.reshape(_ROWS, _COLS)
    sc_out = _sc_partial(a, b)
    tc_out = _tc_partial(a, b)
    return tc_out + jnp.sum(sc_out) * (1.0 / _N)


# speedup vs baseline: 1.0091x; 1.0039x over previous
import functools

import jax
import jax.numpy as jnp
from jax import lax
from jax.experimental import pallas as pl
from jax.experimental.pallas import tpu as pltpu
from jax.experimental.pallas import tpu_sc as plsc

_N = 16 * 3 * 512 * 512
_ROWS = 24576
_COLS = 512

_NC = 2
_NS = 16
_NW = _NC * _NS
_SC_ROWS = 4096
_W_ROWS = _SC_ROWS // _NW
_CH = 32
_NCH = _W_ROWS // _CH
_NBUF = 3

_TC_ROWS = _ROWS - _SC_ROWS
_K = 4
_STEPS = 10
_BR = _TC_ROWS // (_K * _STEPS)
_TC_BASE = _SC_ROWS // _BR


def _tc_kernel(*refs):
    a_refs = refs[:_K]
    b_refs = refs[_K:2 * _K]
    o_ref = refs[2 * _K]
    acc_ref = refs[2 * _K + 1]
    i = pl.program_id(0)

    total = jnp.zeros((8, _COLS), jnp.float32)
    for k in range(_K):
        d = jnp.abs(a_refs[k][...] - b_refs[k][...])
        total = total + jnp.sum(d.reshape(_BR // 8, 8, _COLS), axis=0)

    @pl.when(i == 0)
    def _init():
        acc_ref[...] = total

    @pl.when(i > 0)
    def _acc():
        acc_ref[...] += total

    @pl.when(i == _STEPS - 1)
    def _fin():
        o_ref[0, 0] = jnp.sum(acc_ref[...]) * (1.0 / _N)


def _make_tc_spec(k):
    return pl.BlockSpec(
        (_BR, _COLS), lambda i, k=k: (_TC_BASE + k * _STEPS + i, 0)
    )


def _tc_partial(a, b):
    in_specs = [_make_tc_spec(k) for k in range(_K)] * 2
    out = pl.pallas_call(
        _tc_kernel,
        grid=(_STEPS,),
        in_specs=in_specs,
        out_specs=pl.BlockSpec(
            (1, 1), lambda i: (0, 0), memory_space=pltpu.SMEM
        ),
        out_shape=jax.ShapeDtypeStruct((1, 1), jnp.float32),
        scratch_shapes=[pltpu.VMEM((8, _COLS), jnp.float32)],
    )(*([a] * _K + [b] * _K))
    return out[0, 0]


def _sc_partial(a, b):
    mesh = plsc.VectorSubcoreMesh(
        core_axis_name="c", subcore_axis_name="s",
        num_cores=_NC, num_subcores=_NS,
    )

    buf_types = [pltpu.VMEM((_CH, _COLS), jnp.float32)] * (2 * _NBUF)
    sem_types = [pltpu.SemaphoreType.DMA] * (2 * _NBUF)

    @functools.partial(
        pl.kernel,
        out_type=jax.ShapeDtypeStruct((_NW, 16), jnp.float32),
        mesh=mesh,
        scratch_types=buf_types + [pltpu.VMEM((16,), jnp.float32)] + sem_types,
    )
    def sc_kernel(a_hbm, b_hbm, o_hbm, *rest):
        abufs = rest[:_NBUF]
        bbufs = rest[_NBUF:2 * _NBUF]
        accv = rest[2 * _NBUF]
        asems = rest[2 * _NBUF + 1:2 * _NBUF + 1 + _NBUF]
        bsems = rest[2 * _NBUF + 1 + _NBUF:2 * _NBUF + 1 + 2 * _NBUF]
        wid = lax.axis_index("s") * _NC + lax.axis_index("c")
        base = wid * _W_ROWS

        def start(ci):
            slot = ci % _NBUF
            r0 = base + ci * _CH
            ca = pltpu.async_copy(
                a_hbm.at[pl.ds(r0, _CH)], abufs[slot], asems[slot]
            )
            cb = pltpu.async_copy(
                b_hbm.at[pl.ds(r0, _CH)], bbufs[slot], bsems[slot]
            )
            return ca, cb

        def accum_chunk(a_ref, b_ref, accs):
            def body(r, accs):
                new = list(accs)
                for t in range(32):
                    va = a_ref[r, pl.ds(t * 16, 16)]
                    vb = b_ref[r, pl.ds(t * 16, 16)]
                    new[t % 8] = new[t % 8] + jnp.abs(va - vb)
                return tuple(new)

            return plsc.parallel_loop(0, _CH, 1, unroll=2, carry=accs)(body)

        zero = jnp.zeros((16,), jnp.float32)
        accs = (zero,) * 8
        pending = [start(ci) for ci in range(min(_NBUF - 1, _NCH))]
        for ci in range(_NCH):
            if ci + _NBUF - 1 < _NCH:
                pending.append(start(ci + _NBUF - 1))
            ca, cb = pending.pop(0)
            ca.wait()
            cb.wait()
            slot = ci % _NBUF
            accs = accum_chunk(abufs[slot], bbufs[slot], accs)

        accv[...] = (
            ((accs[0] + accs[1]) + (accs[2] + accs[3]))
            + ((accs[4] + accs[5]) + (accs[6] + accs[7]))
        )
        pltpu.sync_copy(accv, o_hbm.at[wid])

    return sc_kernel(a, b)


def kernel(inp, tar, boxes, texts):
    a = inp.reshape(_ROWS, _COLS)
    b = tar.reshape(_ROWS, _COLS)
    sc_out = _sc_partial(a, b)
    tc_out = _tc_partial(a, b)
    return tc_out + jnp.sum(sc_out) * (1.0 / _N)


# speedup vs baseline: 1.0493x; 1.0399x over previous
import functools

import jax
import jax.numpy as jnp
from jax import lax
from jax.experimental import pallas as pl
from jax.experimental.pallas import tpu as pltpu
from jax.experimental.pallas import tpu_sc as plsc

_N = 16 * 3 * 512 * 512
_ROWS = 24576
_COLS = 512

_NC = 1
_NS = 16
_NW = _NC * _NS
_SC_ROWS = 4096
_W_ROWS = _SC_ROWS // _NW
_CH = 32
_NCH = _W_ROWS // _CH
_NBUF = 3

_TC_ROWS = _ROWS - _SC_ROWS
_K = 4
_STEPS = 10
_BR = _TC_ROWS // (_K * _STEPS)
_TC_BASE = _SC_ROWS // _BR


def _tc_kernel(*refs):
    a_refs = refs[:_K]
    b_refs = refs[_K:2 * _K]
    o_ref = refs[2 * _K]
    acc_ref = refs[2 * _K + 1]
    i = pl.program_id(0)

    total = jnp.zeros((8, _COLS), jnp.float32)
    for k in range(_K):
        d = jnp.abs(a_refs[k][...] - b_refs[k][...])
        total = total + jnp.sum(d.reshape(_BR // 8, 8, _COLS), axis=0)

    @pl.when(i == 0)
    def _init():
        acc_ref[...] = total

    @pl.when(i > 0)
    def _acc():
        acc_ref[...] += total

    @pl.when(i == _STEPS - 1)
    def _fin():
        o_ref[0, 0] = jnp.sum(acc_ref[...]) * (1.0 / _N)


def _make_tc_spec(k):
    return pl.BlockSpec(
        (_BR, _COLS), lambda i, k=k: (_TC_BASE + k * _STEPS + i, 0)
    )


def _tc_partial(a, b):
    in_specs = [_make_tc_spec(k) for k in range(_K)] * 2
    out = pl.pallas_call(
        _tc_kernel,
        grid=(_STEPS,),
        in_specs=in_specs,
        out_specs=pl.BlockSpec(
            (1, 1), lambda i: (0, 0), memory_space=pltpu.SMEM
        ),
        out_shape=jax.ShapeDtypeStruct((1, 1), jnp.float32),
        scratch_shapes=[pltpu.VMEM((8, _COLS), jnp.float32)],
    )(*([a] * _K + [b] * _K))
    return out[0, 0]


def _sc_partial(a, b):
    mesh = plsc.VectorSubcoreMesh(
        core_axis_name="c", subcore_axis_name="s",
        num_cores=_NC, num_subcores=_NS,
    )

    buf_types = [pltpu.VMEM((_CH, _COLS), jnp.float32)] * (2 * _NBUF)
    sem_types = [pltpu.SemaphoreType.DMA] * (2 * _NBUF)

    @functools.partial(
        pl.kernel,
        out_type=jax.ShapeDtypeStruct((_NW, 16), jnp.float32),
        mesh=mesh,
        scratch_types=buf_types + [pltpu.VMEM((16,), jnp.float32)] + sem_types,
    )
    def sc_kernel(a_hbm, b_hbm, o_hbm, *rest):
        abufs = rest[:_NBUF]
        bbufs = rest[_NBUF:2 * _NBUF]
        accv = rest[2 * _NBUF]
        asems = rest[2 * _NBUF + 1:2 * _NBUF + 1 + _NBUF]
        bsems = rest[2 * _NBUF + 1 + _NBUF:2 * _NBUF + 1 + 2 * _NBUF]
        wid = lax.axis_index("s") * _NC + lax.axis_index("c")
        base = wid * _W_ROWS

        def start(ci):
            slot = ci % _NBUF
            r0 = base + ci * _CH
            ca = pltpu.async_copy(
                a_hbm.at[pl.ds(r0, _CH)], abufs[slot], asems[slot]
            )
            cb = pltpu.async_copy(
                b_hbm.at[pl.ds(r0, _CH)], bbufs[slot], bsems[slot]
            )
            return ca, cb

        def accum_chunk(a_ref, b_ref, accs):
            def body(r, accs):
                new = list(accs)
                for t in range(32):
                    va = a_ref[r, pl.ds(t * 16, 16)]
                    vb = b_ref[r, pl.ds(t * 16, 16)]
                    new[t % 8] = new[t % 8] + jnp.abs(va - vb)
                return tuple(new)

            return plsc.parallel_loop(0, _CH, 1, unroll=2, carry=accs)(body)

        zero = jnp.zeros((16,), jnp.float32)
        accs = (zero,) * 8
        pending = [start(ci) for ci in range(min(_NBUF - 1, _NCH))]
        for ci in range(_NCH):
            if ci + _NBUF - 1 < _NCH:
                pending.append(start(ci + _NBUF - 1))
            ca, cb = pending.pop(0)
            ca.wait()
            cb.wait()
            slot = ci % _NBUF
            accs = accum_chunk(abufs[slot], bbufs[slot], accs)

        accv[...] = (
            ((accs[0] + accs[1]) + (accs[2] + accs[3]))
            + ((accs[4] + accs[5]) + (accs[6] + accs[7]))
        )
        pltpu.sync_copy(accv, o_hbm.at[wid])

    return sc_kernel(a, b)


def kernel(inp, tar, boxes, texts):
    a = inp.reshape(_ROWS, _COLS)
    b = tar.reshape(_ROWS, _COLS)
    sc_out = _sc_partial(a, b)
    tc_out = _tc_partial(a, b)
    return tc_out + jnp.sum(sc_out) * (1.0 / _N)


# speedup vs baseline: 1.0595x; 1.0098x over previous
import functools

import jax
import jax.numpy as jnp
from jax import lax
from jax.experimental import pallas as pl
from jax.experimental.pallas import tpu as pltpu
from jax.experimental.pallas import tpu_sc as plsc

_N = 16 * 3 * 512 * 512
_ROWS = 24576
_COLS = 512

_NC = 1
_NS = 16
_NW = _NC * _NS
_SC_ROWS = 4096
_W_ROWS = _SC_ROWS // _NW
_CH = 32
_NCH = _W_ROWS // _CH
_NBUF = 3

_TC_ROWS = _ROWS - _SC_ROWS
_K = 4
_STEPS = 10
_BR = _TC_ROWS // (_K * _STEPS)
_TC_BASE = _SC_ROWS // _BR


def _tc_kernel(*refs):
    a_refs = refs[:_K]
    b_refs = refs[_K:2 * _K]
    o_ref = refs[2 * _K]
    acc_ref = refs[2 * _K + 1]
    i = pl.program_id(0)

    total = jnp.zeros((8, _COLS), jnp.float32)
    for k in range(_K):
        d = jnp.abs(a_refs[k][...] - b_refs[k][...])
        total = total + jnp.sum(d.reshape(_BR // 8, 8, _COLS), axis=0)

    @pl.when(i == 0)
    def _init():
        acc_ref[...] = total

    @pl.when(i > 0)
    def _acc():
        acc_ref[...] += total

    @pl.when(i == _STEPS - 1)
    def _fin():
        o_ref[0, 0] = jnp.sum(acc_ref[...]) * (1.0 / _N)


def _make_tc_spec(k):
    return pl.BlockSpec(
        (_BR, _COLS), lambda i, k=k: (_TC_BASE + k * _STEPS + i, 0)
    )


def _tc_partial(a, b):
    in_specs = [_make_tc_spec(k) for k in range(_K)] * 2
    out = pl.pallas_call(
        _tc_kernel,
        grid=(_STEPS,),
        in_specs=in_specs,
        out_specs=pl.BlockSpec(
            (1, 1), lambda i: (0, 0), memory_space=pltpu.SMEM
        ),
        out_shape=jax.ShapeDtypeStruct((1, 1), jnp.float32),
        scratch_shapes=[pltpu.VMEM((8, _COLS), jnp.float32)],
        cost_estimate=pl.CostEstimate(
            flops=3 * _TC_ROWS * _COLS,
            transcendentals=0,
            bytes_accessed=2 * 4 * _TC_ROWS * _COLS,
        ),
    )(*([a] * _K + [b] * _K))
    return out[0, 0]


def _sc_partial(a, b):
    mesh = plsc.VectorSubcoreMesh(
        core_axis_name="c", subcore_axis_name="s",
        num_cores=_NC, num_subcores=_NS,
    )

    buf_types = [pltpu.VMEM((_CH, _COLS), jnp.float32)] * (2 * _NBUF)
    sem_types = [pltpu.SemaphoreType.DMA] * (2 * _NBUF)

    @functools.partial(
        pl.kernel,
        out_type=jax.ShapeDtypeStruct((_NW, 16), jnp.float32),
        mesh=mesh,
        scratch_types=buf_types + [pltpu.VMEM((16,), jnp.float32)] + sem_types,
        cost_estimate=pl.CostEstimate(
            flops=3 * _SC_ROWS * _COLS,
            transcendentals=0,
            bytes_accessed=2 * 4 * _SC_ROWS * _COLS,
        ),
    )
    def sc_kernel(a_hbm, b_hbm, o_hbm, *rest):
        abufs = rest[:_NBUF]
        bbufs = rest[_NBUF:2 * _NBUF]
        accv = rest[2 * _NBUF]
        asems = rest[2 * _NBUF + 1:2 * _NBUF + 1 + _NBUF]
        bsems = rest[2 * _NBUF + 1 + _NBUF:2 * _NBUF + 1 + 2 * _NBUF]
        wid = lax.axis_index("s") * _NC + lax.axis_index("c")
        base = wid * _W_ROWS

        def start(ci):
            slot = ci % _NBUF
            r0 = base + ci * _CH
            ca = pltpu.async_copy(
                a_hbm.at[pl.ds(r0, _CH)], abufs[slot], asems[slot]
            )
            cb = pltpu.async_copy(
                b_hbm.at[pl.ds(r0, _CH)], bbufs[slot], bsems[slot]
            )
            return ca, cb

        def accum_chunk(a_ref, b_ref, accs):
            def body(r, accs):
                new = list(accs)
                for t in range(32):
                    va = a_ref[r, pl.ds(t * 16, 16)]
                    vb = b_ref[r, pl.ds(t * 16, 16)]
                    new[t % 8] = new[t % 8] + jnp.abs(va - vb)
                return tuple(new)

            return plsc.parallel_loop(0, _CH, 1, unroll=2, carry=accs)(body)

        zero = jnp.zeros((16,), jnp.float32)
        accs = (zero,) * 8
        pending = [start(ci) for ci in range(min(_NBUF - 1, _NCH))]
        for ci in range(_NCH):
            if ci + _NBUF - 1 < _NCH:
                pending.append(start(ci + _NBUF - 1))
            ca, cb = pending.pop(0)
            ca.wait()
            cb.wait()
            slot = ci % _NBUF
            accs = accum_chunk(abufs[slot], bbufs[slot], accs)

        accv[...] = (
            ((accs[0] + accs[1]) + (accs[2] + accs[3]))
            + ((accs[4] + accs[5]) + (accs[6] + accs[7]))
        )
        pltpu.sync_copy(accv, o_hbm.at[wid])

    return sc_kernel(a, b)


def kernel(inp, tar, boxes, texts):
    a = inp.reshape(_ROWS, _COLS)
    b = tar.reshape(_ROWS, _COLS)
    sc_out = _sc_partial(a, b)
    tc_out = _tc_partial(a, b)
    return tc_out + jnp.sum(sc_out) * (1.0 / _N)


# speedup vs baseline: 1.6489x; 1.5562x over previous
import jax
import jax.numpy as jnp
from jax.experimental import pallas as pl
from jax.experimental.pallas import tpu as pltpu

_N = 16 * 3 * 512 * 512
_ROWS = 24576
_COLS = 512
_K = 4
_STEPS = 12
_BR = _ROWS // (_K * _STEPS)


def _l1_mean_kernel(*refs):
    a_refs = refs[:_K]
    b_refs = refs[_K:2 * _K]
    o_ref = refs[2 * _K]
    acc_ref = refs[2 * _K + 1]
    i = pl.program_id(0)

    total = jnp.zeros((8, _COLS), jnp.float32)
    for k in range(_K):
        d = jnp.abs(a_refs[k][...] - b_refs[k][...])
        total = total + jnp.sum(d.reshape(_BR // 8, 8, _COLS), axis=0)

    @pl.when(i == 0)
    def _init():
        acc_ref[...] = total

    @pl.when(i > 0)
    def _acc():
        acc_ref[...] += total

    @pl.when(i == _STEPS - 1)
    def _fin():
        o_ref[0, 0] = jnp.sum(acc_ref[...]) * (1.0 / _N)


def _make_spec(k):
    return pl.BlockSpec((_BR, _COLS), lambda i, k=k: (k * _STEPS + i, 0))


def kernel(inp, tar, boxes, texts):
    a = inp.reshape(_ROWS, _COLS)
    b = tar.reshape(_ROWS, _COLS)
    in_specs = [_make_spec(k) for k in range(_K)] * 2
    out = pl.pallas_call(
        _l1_mean_kernel,
        grid=(_STEPS,),
        in_specs=in_specs,
        out_specs=pl.BlockSpec(
            (1, 1), lambda i: (0, 0), memory_space=pltpu.SMEM
        ),
        out_shape=jax.ShapeDtypeStruct((1, 1), jnp.float32),
        scratch_shapes=[pltpu.VMEM((8, _COLS), jnp.float32)],
    )(*([a] * _K + [b] * _K))
    return out[0, 0]
